# baseline (speedup 1.0000x reference)
import functools

import jax
import jax.numpy as jnp
from jax import lax
from jax.experimental import pallas as pl
from jax.experimental.pallas import tpu as pltpu
from jax.experimental.pallas import tpu_sc as plsc

B, T, C = 128, 32, 10000
ATT = 64
ROWS = B * T
CB = 5000
NCB = C // CB


def _stream_body(xp_ref, w1_ref, b1_ref, w2_ref, tcp_ref, lens_ref,
                 y_ref, vals_ref, acc_ref, xg_ref, logit_ref):
    c = pl.program_id(0)
    t = pl.program_id(1)
    x = xp_ref[0]
    s = jax.nn.sigmoid(x)
    y_ref[...] = s
    partial = lax.dot_general(w1_ref[...], s, (((0,), (0,)), ((), ())),
                              preferred_element_type=jnp.float32)
    code_ids = c * CB + lax.broadcasted_iota(jnp.int32, (CB, B), 0)
    hit = code_ids == tcp_ref[0]
    xgp = jnp.sum(jnp.where(hit, x, 0.0), axis=0, keepdims=True)

    @pl.when(c == 0)
    def _():
        acc_ref[t] = partial
        xg_ref[pl.ds(t, 1), :] = xgp

    @pl.when(c > 0)
    def _():
        acc_ref[t] += partial
        xg_ref[pl.ds(t, 1), :] += xgp

    @pl.when(c == NCB - 1)
    def _():
        e = jnp.tanh(acc_ref[t] + b1_ref[...])
        lg = lax.dot_general(w2_ref[...], e, (((0,), (0,)), ((), ())),
                             preferred_element_type=jnp.float32)
        logit_ref[pl.ds(t, 1), :] = lg

    @pl.when((c == NCB - 1) & (t == T - 1))
    def _():
        t_ids = lax.broadcasted_iota(jnp.int32, (T, B), 0)
        mask = t_ids < lens_ref[...]
        l = jnp.where(mask, logit_ref[...], -30.0)
        p = jnp.exp(l)
        score = p / jnp.sum(p, axis=0, keepdims=True)
        vals_ref[...] = jax.nn.sigmoid(xg_ref[...] + score)


def _make_scatter():
    info = plsc.get_sparse_core_info()
    nw = info.num_cores * info.num_subcores
    per_w = ROWS // nw

    mesh = plsc.VectorSubcoreMesh(core_axis_name="c", subcore_axis_name="s")

    @functools.partial(
        pl.kernel, mesh=mesh, out_type=(),
        scratch_types=[
            pltpu.VMEM((per_w,), jnp.int32),
            pltpu.VMEM((per_w,), jnp.float32),
            pltpu.SemaphoreType.DMA,
        ],
    )
    def scatter(idx_hbm, vals_hbm, y_ref, idx_v, vals_v, sem):
        wid = lax.axis_index("s") * info.num_cores + lax.axis_index("c")
        base = wid * per_w
        pltpu.sync_copy(idx_hbm.at[pl.ds(base, per_w)], idx_v)
        pltpu.sync_copy(vals_hbm.at[pl.ds(base, per_w)], vals_v)
        pltpu.async_copy(vals_v, y_ref.at[idx_v], sem).wait()

    return scatter


_scatter = None


def kernel(x, lens, target_codes, W1, b1, w2):
    global _scatter
    if _scatter is None:
        _scatter = _make_scatter()

    xp = jnp.transpose(x, (1, 2, 0))
    tcp = jnp.transpose(target_codes, (1, 0)).reshape(T, 1, B)
    lens2 = lens.reshape(1, B)

    y2d, vals = pl.pallas_call(
        _stream_body,
        grid=(NCB, T),
        in_specs=[
            pl.BlockSpec((1, CB, B), lambda c, t: (t, c, 0)),
            pl.BlockSpec((CB, ATT), lambda c, t: (c, 0)),
            pl.BlockSpec((ATT, 1), lambda c, t: (0, 0)),
            pl.BlockSpec((ATT, 1), lambda c, t: (0, 0)),
            pl.BlockSpec((1, 1, B), lambda c, t: (t, 0, 0)),
            pl.BlockSpec((1, B), lambda c, t: (0, 0)),
        ],
        out_specs=[
            pl.BlockSpec((CB, B), lambda c, t: (t * NCB + c, 0)),
            pl.BlockSpec((T, B), lambda c, t: (0, 0)),
        ],
        out_shape=[
            jax.ShapeDtypeStruct((T * C, B), jnp.float32),
            jax.ShapeDtypeStruct((T, B), jnp.float32),
        ],
        scratch_shapes=[
            pltpu.VMEM((T, ATT, B), jnp.float32),
            pltpu.VMEM((T, B), jnp.float32),
            pltpu.VMEM((T, B), jnp.float32),
        ],
    )(xp, W1, b1.reshape(ATT, 1), w2.reshape(ATT, 1), tcp, lens2)

    tt = jnp.arange(T, dtype=jnp.int32).reshape(T, 1)
    bb = jnp.arange(B, dtype=jnp.int32).reshape(1, B)
    idx = ((tt * C + jnp.transpose(target_codes, (1, 0))) * B
           + bb).reshape(ROWS)

    y_ref = jax.new_ref(y2d.reshape(T * C * B))
    _scatter(idx, vals.reshape(ROWS), y_ref)
    yf = jax.freeze(y_ref)
    return jnp.transpose(yf.reshape(T, C, B), (2, 0, 1))


# speedup vs baseline: 1.1816x; 1.1816x over previous
import functools

import jax
import jax.numpy as jnp
from jax import lax
from jax.experimental import pallas as pl
from jax.experimental.pallas import tpu as pltpu
from jax.experimental.pallas import tpu_sc as plsc

B, T, C = 128, 32, 10000
ATT = 64
ROWS = B * T
CB = 10000
NCB = C // CB


def _stream_body(xp_ref, w1_ref, b1_ref, w2_ref, lens_ref,
                 y_ref, score_ref, acc_ref, logit_ref):
    c = pl.program_id(0)
    t = pl.program_id(1)
    x = xp_ref[0]
    s = jax.nn.sigmoid(x)
    y_ref[...] = s
    partial = lax.dot_general(w1_ref[...], s, (((0,), (0,)), ((), ())),
                              preferred_element_type=jnp.float32)

    @pl.when(c == 0)
    def _():
        acc_ref[t] = partial

    @pl.when(c > 0)
    def _():
        acc_ref[t] += partial

    @pl.when(c == NCB - 1)
    def _():
        e = jnp.tanh(acc_ref[t] + b1_ref[...])
        lg = lax.dot_general(w2_ref[...], e, (((0,), (0,)), ((), ())),
                             preferred_element_type=jnp.float32)
        logit_ref[pl.ds(t, 1), :] = lg

    @pl.when((c == NCB - 1) & (t == T - 1))
    def _():
        t_ids = lax.broadcasted_iota(jnp.int32, (T, B), 0)
        mask = t_ids < lens_ref[...]
        l = jnp.where(mask, logit_ref[...], -30.0)
        p = jnp.exp(l)
        score_ref[...] = p / jnp.sum(p, axis=0, keepdims=True)


def _make_scatter():
    info = plsc.get_sparse_core_info()
    nw = info.num_cores * info.num_subcores
    per_w = ROWS // nw

    mesh = plsc.VectorSubcoreMesh(core_axis_name="c", subcore_axis_name="s")

    @functools.partial(
        pl.kernel, mesh=mesh, out_type=(),
        scratch_types=[
            pltpu.VMEM((per_w,), jnp.int32),
            pltpu.VMEM((per_w,), jnp.float32),
            pltpu.VMEM((per_w,), jnp.float32),
            pltpu.SemaphoreType.DMA,
        ],
    )
    def fixup(idx_hbm, score_hbm, x_hbm, y_ref, idx_v, xg_v, score_v, sem):
        wid = lax.axis_index("s") * info.num_cores + lax.axis_index("c")
        base = wid * per_w
        pltpu.sync_copy(idx_hbm.at[pl.ds(base, per_w)], idx_v)
        pltpu.sync_copy(score_hbm.at[pl.ds(base, per_w)], score_v)
        pltpu.async_copy(x_hbm.at[idx_v], xg_v, sem).wait()
        for k in range(per_w // 16):
            sl = pl.ds(k * 16, 16)
            z = xg_v[sl] + score_v[sl]
            xg_v[sl] = 1.0 / (1.0 + jnp.exp(-z))
        pltpu.async_copy(xg_v, y_ref.at[idx_v], sem).wait()

    return fixup


_scatter = None


def kernel(x, lens, target_codes, W1, b1, w2):
    global _scatter
    if _scatter is None:
        _scatter = _make_scatter()

    xp = jnp.transpose(x, (1, 2, 0))
    lens2 = lens.reshape(1, B)

    y2d, score = pl.pallas_call(
        _stream_body,
        grid=(NCB, T),
        in_specs=[
            pl.BlockSpec((1, CB, B), lambda c, t: (t, c, 0)),
            pl.BlockSpec((CB, ATT), lambda c, t: (c, 0)),
            pl.BlockSpec((ATT, 1), lambda c, t: (0, 0)),
            pl.BlockSpec((ATT, 1), lambda c, t: (0, 0)),
            pl.BlockSpec((1, B), lambda c, t: (0, 0)),
        ],
        out_specs=[
            pl.BlockSpec((CB, B), lambda c, t: (t * NCB + c, 0)),
            pl.BlockSpec((T, B), lambda c, t: (0, 0)),
        ],
        out_shape=[
            jax.ShapeDtypeStruct((T * C, B), jnp.float32),
            jax.ShapeDtypeStruct((T, B), jnp.float32),
        ],
        scratch_shapes=[
            pltpu.VMEM((T, ATT, B), jnp.float32),
            pltpu.VMEM((T, B), jnp.float32),
        ],
    )(xp, W1, b1.reshape(ATT, 1), w2.reshape(ATT, 1), lens2)

    tt = jnp.arange(T, dtype=jnp.int32).reshape(T, 1)
    bb = jnp.arange(B, dtype=jnp.int32).reshape(1, B)
    idx = ((tt * C + jnp.transpose(target_codes, (1, 0))) * B
           + bb).reshape(ROWS)

    y_ref = jax.new_ref(y2d.reshape(T * C * B))
    _scatter(idx, score.reshape(ROWS), xp.reshape(T * C * B), y_ref)
    yf = jax.freeze(y_ref)
    return jnp.transpose(yf.reshape(T, C, B), (2, 0, 1))


# speedup vs baseline: 1.2114x; 1.0252x over previous
import functools

import jax
import jax.numpy as jnp
from jax import lax
from jax.experimental import pallas as pl
from jax.experimental.pallas import tpu as pltpu
from jax.experimental.pallas import tpu_sc as plsc

B, T, C = 128, 32, 10000
ATT = 64
ROWS = B * T
CB = 10000
NCB = C // CB


TPB = 2
NT = T // TPB


def _stream_body(xp_ref, w1_ref, b1_ref, w2_ref, lens_ref,
                 y_ref, score_ref, logit_ref):
    ts = pl.program_id(0)
    x3 = xp_ref[...]
    s3 = jax.nn.sigmoid(x3)
    y_ref[...] = s3.reshape(TPB * C, B)
    for j in range(TPB):
        acc = lax.dot_general(w1_ref[...], s3[j], (((0,), (0,)), ((), ())),
                              preferred_element_type=jnp.float32)
        e = jnp.tanh(acc + b1_ref[...])
        lg = lax.dot_general(w2_ref[...], e, (((0,), (0,)), ((), ())),
                             preferred_element_type=jnp.float32)
        logit_ref[pl.ds(ts * TPB + j, 1), :] = lg

    @pl.when(ts == NT - 1)
    def _():
        t_ids = lax.broadcasted_iota(jnp.int32, (T, B), 0)
        mask = t_ids < lens_ref[...]
        l = jnp.where(mask, logit_ref[...], -30.0)
        p = jnp.exp(l)
        score_ref[...] = p / jnp.sum(p, axis=0, keepdims=True)


def _make_scatter():
    info = plsc.get_sparse_core_info()
    nw = info.num_cores * info.num_subcores
    per_w = ROWS // nw

    mesh = plsc.VectorSubcoreMesh(core_axis_name="c", subcore_axis_name="s")

    @functools.partial(
        pl.kernel, mesh=mesh, out_type=(),
        scratch_types=[
            pltpu.VMEM((per_w,), jnp.int32),
            pltpu.VMEM((per_w,), jnp.float32),
            pltpu.VMEM((per_w,), jnp.float32),
            pltpu.SemaphoreType.DMA,
        ],
    )
    def fixup(idx_hbm, score_hbm, x_hbm, y_ref, idx_v, xg_v, score_v, sem):
        wid = lax.axis_index("s") * info.num_cores + lax.axis_index("c")
        base = wid * per_w
        pltpu.sync_copy(idx_hbm.at[pl.ds(base, per_w)], idx_v)
        pltpu.sync_copy(score_hbm.at[pl.ds(base, per_w)], score_v)
        pltpu.async_copy(x_hbm.at[idx_v], xg_v, sem).wait()
        for k in range(per_w // 16):
            sl = pl.ds(k * 16, 16)
            z = xg_v[sl] + score_v[sl]
            xg_v[sl] = 1.0 / (1.0 + jnp.exp(-z))
        pltpu.async_copy(xg_v, y_ref.at[idx_v], sem).wait()

    return fixup


_scatter = None


def kernel(x, lens, target_codes, W1, b1, w2):
    global _scatter
    if _scatter is None:
        _scatter = _make_scatter()

    xp = jnp.transpose(x, (1, 2, 0))
    lens2 = lens.reshape(1, B)

    y2d, score = pl.pallas_call(
        _stream_body,
        grid=(NT,),
        in_specs=[
            pl.BlockSpec((TPB, C, B), lambda ts: (ts, 0, 0)),
            pl.BlockSpec((C, ATT), lambda ts: (0, 0)),
            pl.BlockSpec((ATT, 1), lambda ts: (0, 0)),
            pl.BlockSpec((ATT, 1), lambda ts: (0, 0)),
            pl.BlockSpec((1, B), lambda ts: (0, 0)),
        ],
        out_specs=[
            pl.BlockSpec((TPB * C, B), lambda ts: (ts, 0)),
            pl.BlockSpec((T, B), lambda ts: (0, 0)),
        ],
        out_shape=[
            jax.ShapeDtypeStruct((T * C, B), jnp.float32),
            jax.ShapeDtypeStruct((T, B), jnp.float32),
        ],
        scratch_shapes=[
            pltpu.VMEM((T, B), jnp.float32),
        ],
    )(xp, W1, b1.reshape(ATT, 1), w2.reshape(ATT, 1), lens2)

    tt = jnp.arange(T, dtype=jnp.int32).reshape(T, 1)
    bb = jnp.arange(B, dtype=jnp.int32).reshape(1, B)
    idx = ((tt * C + jnp.transpose(target_codes, (1, 0))) * B
           + bb).reshape(ROWS)

    y_ref = jax.new_ref(y2d.reshape(T * C * B))
    _scatter(idx, score.reshape(ROWS), xp.reshape(T * C * B), y_ref)
    yf = jax.freeze(y_ref)
    return jnp.transpose(yf.reshape(T, C, B), (2, 0, 1))


# speedup vs baseline: 1.2233x; 1.0098x over previous
import functools

import jax
import jax.numpy as jnp
from jax import lax
from jax.experimental import pallas as pl
from jax.experimental.pallas import tpu as pltpu
from jax.experimental.pallas import tpu_sc as plsc

B, T, C = 128, 32, 10000
ATT = 64
ROWS = B * T
CB = 10000
NCB = C // CB


TPB = 2
NT = T // TPB


def _stream_body(xp_ref, w1_ref, b1_ref, w2_ref, lens_ref,
                 y_ref, score_ref, logit_ref):
    ts = pl.program_id(0)
    x3 = xp_ref[...]
    s3 = jax.nn.sigmoid(x3)
    y_ref[...] = s3.reshape(TPB * C, B)
    for j in range(TPB):
        acc = lax.dot_general(w1_ref[...], s3[j].astype(jnp.bfloat16),
                              (((0,), (0,)), ((), ())),
                              preferred_element_type=jnp.float32)
        e = jnp.tanh(acc + b1_ref[...])
        lg = lax.dot_general(w2_ref[...], e, (((0,), (0,)), ((), ())),
                             preferred_element_type=jnp.float32)
        logit_ref[pl.ds(ts * TPB + j, 1), :] = lg

    @pl.when(ts == NT - 1)
    def _():
        t_ids = lax.broadcasted_iota(jnp.int32, (T, B), 0)
        mask = t_ids < lens_ref[...]
        l = jnp.where(mask, logit_ref[...], -30.0)
        p = jnp.exp(l)
        score_ref[...] = p / jnp.sum(p, axis=0, keepdims=True)


def _make_scatter():
    info = plsc.get_sparse_core_info()
    nw = info.num_cores * info.num_subcores
    per_w = ROWS // nw

    mesh = plsc.VectorSubcoreMesh(core_axis_name="c", subcore_axis_name="s")

    @functools.partial(
        pl.kernel, mesh=mesh, out_type=(),
        scratch_types=[
            pltpu.VMEM((per_w,), jnp.int32),
            pltpu.VMEM((per_w,), jnp.float32),
            pltpu.VMEM((per_w,), jnp.float32),
            pltpu.SemaphoreType.DMA,
        ],
    )
    def fixup(idx_hbm, score_hbm, x_hbm, y_ref, idx_v, xg_v, score_v, sem):
        wid = lax.axis_index("s") * info.num_cores + lax.axis_index("c")
        base = wid * per_w
        pltpu.sync_copy(idx_hbm.at[pl.ds(base, per_w)], idx_v)
        pltpu.sync_copy(score_hbm.at[pl.ds(base, per_w)], score_v)
        pltpu.async_copy(x_hbm.at[idx_v], xg_v, sem).wait()
        for k in range(per_w // 16):
            sl = pl.ds(k * 16, 16)
            z = xg_v[sl] + score_v[sl]
            xg_v[sl] = 1.0 / (1.0 + jnp.exp(-z))
        pltpu.async_copy(xg_v, y_ref.at[idx_v], sem).wait()

    return fixup


_scatter = None


def kernel(x, lens, target_codes, W1, b1, w2):
    global _scatter
    if _scatter is None:
        _scatter = _make_scatter()

    xp = jnp.transpose(x, (1, 2, 0))
    lens2 = lens.reshape(1, B)

    y2d, score = pl.pallas_call(
        _stream_body,
        grid=(NT,),
        in_specs=[
            pl.BlockSpec((TPB, C, B), lambda ts: (ts, 0, 0)),
            pl.BlockSpec((C, ATT), lambda ts: (0, 0)),
            pl.BlockSpec((ATT, 1), lambda ts: (0, 0)),
            pl.BlockSpec((ATT, 1), lambda ts: (0, 0)),
            pl.BlockSpec((1, B), lambda ts: (0, 0)),
        ],
        out_specs=[
            pl.BlockSpec((TPB * C, B), lambda ts: (ts, 0)),
            pl.BlockSpec((T, B), lambda ts: (0, 0)),
        ],
        out_shape=[
            jax.ShapeDtypeStruct((T * C, B), jnp.float32),
            jax.ShapeDtypeStruct((T, B), jnp.float32),
        ],
        scratch_shapes=[
            pltpu.VMEM((T, B), jnp.float32),
        ],
    )(xp, W1.astype(jnp.bfloat16), b1.reshape(ATT, 1),
      w2.reshape(ATT, 1), lens2)

    tt = jnp.arange(T, dtype=jnp.int32).reshape(T, 1)
    bb = jnp.arange(B, dtype=jnp.int32).reshape(1, B)
    idx = ((tt * C + jnp.transpose(target_codes, (1, 0))) * B
           + bb).reshape(ROWS)

    y_ref = jax.new_ref(y2d.reshape(T * C * B))
    _scatter(idx, score.reshape(ROWS), xp.reshape(T * C * B), y_ref)
    yf = jax.freeze(y_ref)
    return jnp.transpose(yf.reshape(T, C, B), (2, 0, 1))
